# baseline (speedup 1.0000x reference)
import functools

import jax
import jax.numpy as jnp
from jax import lax
from jax.experimental import pallas as pl
from jax.experimental.pallas import tpu as pltpu
from jax.experimental.pallas import tpu_sc as plsc

B, H, T_CACHE, D = 8, 16, 4096, 128
T_NEW = 2048
_ROWS = B * H * T_NEW

_NC = 2
_NS = 16
_NW = _NC * _NS
_W_ROWS = _ROWS // _NW
_CHUNK = 256
_NCHUNKS = _W_ROWS // _CHUNK


@functools.partial(
    pl.kernel,
    out_type=[
        jax.ShapeDtypeStruct((_ROWS, D), jnp.float32),
        jax.ShapeDtypeStruct((_ROWS, D), jnp.float32),
        jax.ShapeDtypeStruct((B, T_CACHE), jnp.int32),
    ],
    mesh=plsc.VectorSubcoreMesh(core_axis_name="c", subcore_axis_name="s"),
    scratch_types=[
        pltpu.VMEM((_CHUNK, D), jnp.float32),
        pltpu.VMEM((_CHUNK, D), jnp.float32),
        pltpu.VMEM((T_NEW,), jnp.int32),
        pltpu.SemaphoreType.DMA,
        pltpu.SemaphoreType.DMA,
        pltpu.SemaphoreType.DMA,
        pltpu.SemaphoreType.DMA,
    ],
)
def _sc_all(kv_hbm, vv_hbm, ip_hbm, pos_in_hbm,
            k_out_hbm, v_out_hbm, pos_out_hbm,
            buf0, buf1, ibuf, si0, si1, so0, so1):
    wid = lax.axis_index("s") * _NC + lax.axis_index("c")
    base = wid * _W_ROWS
    bufs = (buf0, buf1)
    sin = (si0, si1)
    sout = (so0, so1)

    for src, dst in ((kv_hbm, k_out_hbm), (vv_hbm, v_out_hbm)):
        out_handles = [None, None]
        for j in range(_NCHUNKS):
            b = j % 2
            if out_handles[b] is not None:
                out_handles[b].wait()
            off = base + j * _CHUNK
            pltpu.async_copy(src.at[pl.ds(off, _CHUNK)], bufs[b],
                             sin[b]).wait()
            out_handles[b] = pltpu.async_copy(
                bufs[b], dst.at[pl.ds(off, _CHUNK)], sout[b])
        for h in out_handles:
            if h is not None:
                h.wait()

    row = wid % B

    @pl.when(wid < B)
    def _():
        pltpu.sync_copy(ip_hbm, ibuf)
        pltpu.sync_copy(ibuf, pos_out_hbm.at[row, pl.ds(0, T_NEW)])

    @pl.when(jnp.logical_and(wid >= B, wid < 2 * B))
    def _():
        pltpu.sync_copy(pos_in_hbm.at[row, pl.ds(T_NEW, T_CACHE - T_NEW)],
                        ibuf)
        pltpu.sync_copy(ibuf, pos_out_hbm.at[row, pl.ds(T_NEW, T_CACHE - T_NEW)])


def kernel(input_pos, k_val, v_val, k_cache, v_cache, pos):
    ip = input_pos.astype(jnp.int32)
    pos2d = pos.reshape(B, T_CACHE)
    kv2 = k_val.reshape(_ROWS, D)
    vv2 = v_val.reshape(_ROWS, D)

    k_out, v_out, pos_out = _sc_all(kv2, vv2, ip, pos2d)

    k = k_out.reshape(B, H, T_NEW, D)
    v = v_out.reshape(B, H, T_NEW, D)
    return (k, v, pos_out.reshape(B, 1, T_CACHE))


# speedup vs baseline: 1.0897x; 1.0897x over previous
import functools

import jax
import jax.numpy as jnp
from jax import lax
from jax.experimental import pallas as pl
from jax.experimental.pallas import tpu as pltpu
from jax.experimental.pallas import tpu_sc as plsc

B, H, T_CACHE, D = 8, 16, 4096, 128
T_NEW = 2048
_ROWS = B * H * T_NEW
_BM = 8192
_GRID = _ROWS // _BM

_NC = 2
_NS = 16
_NW = _NC * _NS
_W_ROWS = _ROWS // _NW
_CHUNK = 256
_NCHUNKS = _W_ROWS // _CHUNK


def _k_body(kv_ref, k_out_ref):
    k_out_ref[...] = kv_ref[...]


def _copy_k(kv2):
    return pl.pallas_call(
        _k_body,
        grid=(_GRID,),
        in_specs=[pl.BlockSpec((_BM, D), lambda i: (i, 0))],
        out_specs=pl.BlockSpec((_BM, D), lambda i: (i, 0)),
        out_shape=jax.ShapeDtypeStruct((_ROWS, D), kv2.dtype),
        compiler_params=pltpu.CompilerParams(
            dimension_semantics=("arbitrary",),
        ),
    )(kv2)


@functools.partial(
    pl.kernel,
    out_type=[
        jax.ShapeDtypeStruct((_ROWS, D), jnp.float32),
        jax.ShapeDtypeStruct((B, T_CACHE), jnp.int32),
    ],
    mesh=plsc.VectorSubcoreMesh(core_axis_name="c", subcore_axis_name="s"),
    scratch_types=[
        pltpu.VMEM((_CHUNK, D), jnp.float32),
        pltpu.VMEM((_CHUNK, D), jnp.float32),
        pltpu.VMEM((T_NEW,), jnp.int32),
        pltpu.SemaphoreType.DMA,
        pltpu.SemaphoreType.DMA,
        pltpu.SemaphoreType.DMA,
        pltpu.SemaphoreType.DMA,
    ],
)
def _sc_v_pos(vv_hbm, ip_hbm, pos_in_hbm,
              v_out_hbm, pos_out_hbm,
              buf0, buf1, ibuf, si0, si1, so0, so1):
    wid = lax.axis_index("s") * _NC + lax.axis_index("c")
    base = wid * _W_ROWS
    bufs = (buf0, buf1)
    sin = (si0, si1)
    sout = (so0, so1)

    out_handles = [None, None]
    for j in range(_NCHUNKS):
        b = j % 2
        if out_handles[b] is not None:
            out_handles[b].wait()
        off = base + j * _CHUNK
        pltpu.async_copy(vv_hbm.at[pl.ds(off, _CHUNK)], bufs[b],
                         sin[b]).wait()
        out_handles[b] = pltpu.async_copy(
            bufs[b], v_out_hbm.at[pl.ds(off, _CHUNK)], sout[b])
    for h in out_handles:
        if h is not None:
            h.wait()

    row = wid % B

    @pl.when(wid < B)
    def _():
        pltpu.sync_copy(ip_hbm, ibuf)
        pltpu.sync_copy(ibuf, pos_out_hbm.at[row, pl.ds(0, T_NEW)])

    @pl.when(jnp.logical_and(wid >= B, wid < 2 * B))
    def _():
        pltpu.sync_copy(pos_in_hbm.at[row, pl.ds(T_NEW, T_CACHE - T_NEW)],
                        ibuf)
        pltpu.sync_copy(ibuf, pos_out_hbm.at[row, pl.ds(T_NEW, T_CACHE - T_NEW)])


def kernel(input_pos, k_val, v_val, k_cache, v_cache, pos):
    ip = input_pos.astype(jnp.int32)
    pos2d = pos.reshape(B, T_CACHE)
    kv2 = k_val.reshape(_ROWS, D)
    vv2 = v_val.reshape(_ROWS, D)

    v_out, pos_out = _sc_v_pos(vv2, ip, pos2d)
    k_out = _copy_k(kv2)

    k = k_out.reshape(B, H, T_NEW, D)
    v = v_out.reshape(B, H, T_NEW, D)
    return (k, v, pos_out.reshape(B, 1, T_CACHE))


# speedup vs baseline: 1.1503x; 1.0556x over previous
import functools

import jax
import jax.numpy as jnp
from jax import lax
from jax.experimental import pallas as pl
from jax.experimental.pallas import tpu as pltpu
from jax.experimental.pallas import tpu_sc as plsc

B, H, T_CACHE, D = 8, 16, 4096, 128
T_NEW = 2048
_ROWS = B * H * T_NEW
_BM = 8192
_GRID = _ROWS // _BM

_NC = 2
_NS = 16


def _kv_body(kv_ref, vv_ref, k_out_ref, v_out_ref):
    k_out_ref[...] = kv_ref[...]
    v_out_ref[...] = vv_ref[...]


def _copy_kv(kv2, vv2):
    return pl.pallas_call(
        _kv_body,
        grid=(_GRID,),
        in_specs=[
            pl.BlockSpec((_BM, D), lambda i: (i, 0)),
            pl.BlockSpec((_BM, D), lambda i: (i, 0)),
        ],
        out_specs=[
            pl.BlockSpec((_BM, D), lambda i: (i, 0)),
            pl.BlockSpec((_BM, D), lambda i: (i, 0)),
        ],
        out_shape=[
            jax.ShapeDtypeStruct((_ROWS, D), kv2.dtype),
            jax.ShapeDtypeStruct((_ROWS, D), vv2.dtype),
        ],
        compiler_params=pltpu.CompilerParams(
            dimension_semantics=("arbitrary",),
        ),
    )(kv2, vv2)


@functools.partial(
    pl.kernel,
    out_type=jax.ShapeDtypeStruct((B, T_CACHE), jnp.int32),
    mesh=plsc.VectorSubcoreMesh(core_axis_name="c", subcore_axis_name="s"),
    scratch_types=[pltpu.VMEM((T_NEW,), jnp.int32)],
)
def _pos_kernel(ip_hbm, pos_in_hbm, out_hbm, ibuf):
    wid = lax.axis_index("s") * _NC + lax.axis_index("c")
    row = wid % B

    @pl.when(wid < B)
    def _():
        pltpu.sync_copy(ip_hbm, ibuf)
        pltpu.sync_copy(ibuf, out_hbm.at[row, pl.ds(0, T_NEW)])

    @pl.when(jnp.logical_and(wid >= B, wid < 2 * B))
    def _():
        pltpu.sync_copy(pos_in_hbm.at[row, pl.ds(T_NEW, T_CACHE - T_NEW)],
                        ibuf)
        pltpu.sync_copy(ibuf, out_hbm.at[row, pl.ds(T_NEW, T_CACHE - T_NEW)])


def kernel(input_pos, k_val, v_val, k_cache, v_cache, pos):
    ip = input_pos.astype(jnp.int32)
    pos2d = pos.reshape(B, T_CACHE)
    kv2 = k_val.reshape(_ROWS, D)
    vv2 = v_val.reshape(_ROWS, D)

    pos_out = _pos_kernel(ip, pos2d)
    k_out, v_out = _copy_kv(kv2, vv2)

    k = k_out.reshape(B, H, T_NEW, D)
    v = v_out.reshape(B, H, T_NEW, D)
    return (k, v, pos_out.reshape(B, 1, T_CACHE))


# speedup vs baseline: 1.1547x; 1.0039x over previous
import functools

import jax
import jax.numpy as jnp
from jax import lax
from jax.experimental import pallas as pl
from jax.experimental.pallas import tpu as pltpu
from jax.experimental.pallas import tpu_sc as plsc

B, H, T_CACHE, D = 8, 16, 4096, 128
T_NEW = 2048
_ROWS = B * H * T_NEW
_BM = 8192
_GRID = _ROWS // _BM

_NC = 2
_NS = 16


_BM1 = 16384


def _one_body(src_ref, dst_ref):
    dst_ref[...] = src_ref[...]


def _copy_one(x2):
    return pl.pallas_call(
        _one_body,
        grid=(_ROWS // _BM1,),
        in_specs=[pl.BlockSpec((_BM1, D), lambda i: (i, 0))],
        out_specs=pl.BlockSpec((_BM1, D), lambda i: (i, 0)),
        out_shape=jax.ShapeDtypeStruct((_ROWS, D), x2.dtype),
        compiler_params=pltpu.CompilerParams(
            dimension_semantics=("arbitrary",),
        ),
    )(x2)


def _copy_kv(kv2, vv2):
    return _copy_one(kv2), _copy_one(vv2)


@functools.partial(
    pl.kernel,
    out_type=jax.ShapeDtypeStruct((B, T_CACHE), jnp.int32),
    mesh=plsc.VectorSubcoreMesh(core_axis_name="c", subcore_axis_name="s"),
    scratch_types=[pltpu.VMEM((T_NEW,), jnp.int32)],
)
def _pos_kernel(ip_hbm, pos_in_hbm, out_hbm, ibuf):
    wid = lax.axis_index("s") * _NC + lax.axis_index("c")
    row = wid % B

    @pl.when(wid < B)
    def _():
        pltpu.sync_copy(ip_hbm, ibuf)
        pltpu.sync_copy(ibuf, out_hbm.at[row, pl.ds(0, T_NEW)])

    @pl.when(jnp.logical_and(wid >= B, wid < 2 * B))
    def _():
        pltpu.sync_copy(pos_in_hbm.at[row, pl.ds(T_NEW, T_CACHE - T_NEW)],
                        ibuf)
        pltpu.sync_copy(ibuf, out_hbm.at[row, pl.ds(T_NEW, T_CACHE - T_NEW)])


def kernel(input_pos, k_val, v_val, k_cache, v_cache, pos):
    ip = input_pos.astype(jnp.int32)
    pos2d = pos.reshape(B, T_CACHE)
    kv2 = k_val.reshape(_ROWS, D)
    vv2 = v_val.reshape(_ROWS, D)

    pos_out = _pos_kernel(ip, pos2d)
    k_out, v_out = _copy_kv(kv2, vv2)

    k = k_out.reshape(B, H, T_NEW, D)
    v = v_out.reshape(B, H, T_NEW, D)
    return (k, v, pos_out.reshape(B, 1, T_CACHE))


# speedup vs baseline: 1.1551x; 1.0004x over previous
import functools

import jax
import jax.numpy as jnp
from jax import lax
from jax.experimental import pallas as pl
from jax.experimental.pallas import tpu as pltpu
from jax.experimental.pallas import tpu_sc as plsc

B, H, T_CACHE, D = 8, 16, 4096, 128
T_NEW = 2048
_ROWS = B * H * T_NEW
_BM = 8192
_GRID = _ROWS // _BM

_NC = 2
_NS = 16


_BM1 = 16384


def _one_body(src_ref, dst_ref):
    dst_ref[...] = src_ref[...]


def _copy_one(x2):
    return pl.pallas_call(
        _one_body,
        grid=(_ROWS // _BM1,),
        in_specs=[pl.BlockSpec((_BM1, D), lambda i: (i, 0))],
        out_specs=pl.BlockSpec((_BM1, D), lambda i: (i, 0)),
        out_shape=jax.ShapeDtypeStruct((_ROWS, D), x2.dtype),
        compiler_params=pltpu.CompilerParams(
            dimension_semantics=("arbitrary",),
        ),
    )(x2)


def _copy_kv(kv2, vv2):
    return _copy_one(kv2), _copy_one(vv2)


@functools.partial(
    pl.kernel,
    out_type=jax.ShapeDtypeStruct((B, T_CACHE), jnp.int32),
    mesh=plsc.VectorSubcoreMesh(core_axis_name="c", subcore_axis_name="s"),
    scratch_types=[pltpu.VMEM((T_NEW,), jnp.int32)],
)
def _pos_kernel(ip_hbm, pos_in_hbm, out_hbm, ibuf):
    wid = lax.axis_index("s") * _NC + lax.axis_index("c")
    row = wid % B

    @pl.when(wid < B)
    def _():
        pltpu.sync_copy(ip_hbm, ibuf)
        pltpu.sync_copy(ibuf, out_hbm.at[row, pl.ds(0, T_NEW)])

    @pl.when(jnp.logical_and(wid >= B, wid < 2 * B))
    def _():
        pltpu.sync_copy(pos_in_hbm.at[row, pl.ds(T_NEW, T_CACHE - T_NEW)],
                        ibuf)
        pltpu.sync_copy(ibuf, out_hbm.at[row, pl.ds(T_NEW, T_CACHE - T_NEW)])


def kernel(input_pos, k_val, v_val, k_cache, v_cache, pos):
    ip = input_pos.astype(jnp.int32)
    pos2d = pos.reshape(B, T_CACHE)
    kv2 = k_val.reshape(_ROWS, D)
    vv2 = v_val.reshape(_ROWS, D)

    k_out = _copy_one(kv2)
    pos_out = _pos_kernel(ip, pos2d)
    v_out = _copy_one(vv2)

    k = k_out.reshape(B, H, T_NEW, D)
    v = v_out.reshape(B, H, T_NEW, D)
    return (k, v, pos_out.reshape(B, 1, T_CACHE))
